# baseline (speedup 1.0000x reference)
import functools

import jax
import jax.numpy as jnp
from jax import lax
from jax.experimental import pallas as pl
from jax.experimental.pallas import tpu as pltpu
from jax.experimental.pallas import tpu_sc as plsc


def _make_sc_gather(n_rows_total, dim, chunk, n_chunks_per_worker, n_workers,
                    n_cores):
    mesh = plsc.VectorSubcoreMesh(core_axis_name="c", subcore_axis_name="s")
    rows_per_worker = chunk * n_chunks_per_worker

    @functools.partial(
        pl.kernel,
        out_type=jax.ShapeDtypeStruct((n_rows_total, dim), jnp.float32),
        mesh=mesh,
        scratch_types=[
            pltpu.VMEM((n_chunks_per_worker, chunk), jnp.int32),
            pltpu.VMEM((chunk, dim), jnp.float32),
            pltpu.VMEM((chunk, dim), jnp.float32),
            pltpu.SemaphoreType.DMA,
        ],
        compiler_params=pltpu.CompilerParams(use_tc_tiling_on_sc=False),
    )
    def sc_kernel(ids_hbm, table_hbm, pat_hbm, out_hbm, idx_v, pat_v, rows_v,
                  sem):
        wid = lax.axis_index("s") * n_cores + lax.axis_index("c")
        base = wid * rows_per_worker
        pltpu.sync_copy(ids_hbm.at[wid], idx_v)
        pltpu.sync_copy(pat_hbm, pat_v)

        def chunk_body(g, carry):
            pltpu.async_copy(table_hbm.at[idx_v.at[g]], rows_v, sem).wait()

            def add_body(r, c2):
                plsc.addupdate(rows_v.at[r, pl.ds(0, 16)],
                               pat_v[r, pl.ds(0, 16)])
                plsc.addupdate(rows_v.at[r, pl.ds(16, 16)],
                               pat_v[r, pl.ds(16, 16)])
                return c2

            lax.fori_loop(0, chunk, add_body, 0, unroll=4)

            pltpu.sync_copy(rows_v, out_hbm.at[pl.ds(base + g * chunk, chunk)])
            return carry

        lax.fori_loop(0, n_chunks_per_worker, chunk_body, 0)

    return sc_kernel


def kernel(cat_ids, cond_table, cat_table):
    b, n_cond = cat_ids.shape
    dim = cat_table.shape[1]
    n_cat = cat_table.shape[0] // n_cond

    info = plsc.get_sparse_core_info()
    n_cores, n_subcores = info.num_cores, info.num_subcores
    n_workers = n_cores * n_subcores

    n_rows = b * n_cond
    chunk = 4 * n_cond
    rows_per_worker = n_rows // n_workers
    n_chunks_per_worker = rows_per_worker // chunk
    assert rows_per_worker % chunk == 0

    offsets = jnp.arange(n_cond, dtype=jnp.int32) * n_cat
    flat_ids = (cat_ids.astype(jnp.int32) + offsets[None, :]).reshape(
        n_workers, n_chunks_per_worker, chunk)
    pat = jnp.tile(cond_table[1:n_cond + 1], (chunk // n_cond, 1))

    sc_gather = _make_sc_gather(n_rows, dim, chunk, n_chunks_per_worker,
                                n_workers, n_cores)
    out = sc_gather(flat_ids, cat_table, pat)
    return out.reshape(b, n_cond, dim)
